# Initial kernel scaffold; baseline (speedup 1.0000x reference)
import jax
import jax.numpy as jnp
from jax.experimental import pallas as pl


def kernel(x, edge_index, edge_weight, W1, b1, W2, b2):
    raise NotImplementedError("write your pallas kernel here")


# speedup vs baseline: 15.1621x; 15.1621x over previous
import functools

import jax
import jax.numpy as jnp
from jax import lax
from jax.experimental import pallas as pl
from jax.experimental.pallas import tpu as pltpu
from jax.experimental.pallas import tpu_sc as plsc

N = 10000
E = 320000
NTILES = 32
CHUNK = 128
NCHUNK = 80
EPT = CHUNK * NCHUNK
N_PAD = 10240

_mesh = plsc.VectorSubcoreMesh(core_axis_name="c", subcore_axis_name="s")
_sc_params = pltpu.CompilerParams(use_tc_tiling_on_sc=False)


@functools.partial(
    pl.kernel,
    out_type=jax.ShapeDtypeStruct((2, N_PAD), jnp.float32),
    mesh=_mesh,
    compiler_params=_sc_params,
    scratch_types=[
        pltpu.VMEM((NCHUNK, CHUNK), jnp.int32),
        pltpu.VMEM((NCHUNK, CHUNK), jnp.float32),
        pltpu.VMEM_SHARED((N_PAD,), jnp.float32),
    ],
)
def _sc_degree(dstp, ewp, zeros1d, out, dst_v, ew_v, acc):
    cid = lax.axis_index("c")
    sid = lax.axis_index("s")
    wid = cid * 16 + sid
    stripe = pl.ds(sid * 640, 640)
    pltpu.sync_copy(zeros1d.at[stripe], acc.at[stripe])
    pltpu.sync_copy(dstp.at[wid], dst_v)
    pltpu.sync_copy(ewp.at[wid], ew_v)
    plsc.subcore_barrier()

    def body(j, _):
        pltpu.sync_copy(ew_v.at[j], acc.at[dst_v.at[j]], add=True)
        return _

    lax.fori_loop(0, NCHUNK, body, None)
    plsc.subcore_barrier()
    pltpu.sync_copy(acc.at[stripe], out.at[cid, stripe])


def _make_sc_aggregate(d):
    nvec = d // 16

    @functools.partial(
        pl.kernel,
        out_type=jax.ShapeDtypeStruct((2, N_PAD, d), jnp.float32),
        mesh=_mesh,
        compiler_params=_sc_params,
        scratch_types=[
            pltpu.VMEM((NCHUNK, CHUNK), jnp.int32),
            pltpu.VMEM((NCHUNK, CHUNK), jnp.int32),
            pltpu.VMEM((NCHUNK, CHUNK), jnp.float32),
            pltpu.VMEM((CHUNK, d), jnp.float32),
            pltpu.VMEM_SHARED((N_PAD, d), jnp.float32),
            pltpu.SemaphoreType.DMA,
        ],
    )
    def agg(hs, srcp, dstp, ewp, zeros2d, out, src_v, dst_v, ew_v, rows_v,
            acc, sem):
        cid = lax.axis_index("c")
        sid = lax.axis_index("s")
        wid = cid * 16 + sid
        stripe = pl.ds(sid * 640, 640)
        pltpu.sync_copy(zeros2d.at[stripe], acc.at[stripe])
        pltpu.sync_copy(srcp.at[wid], src_v)
        pltpu.sync_copy(dstp.at[wid], dst_v)
        pltpu.sync_copy(ewp.at[wid], ew_v)
        plsc.subcore_barrier()

        def chunk_body(j, _):
            pltpu.async_copy(hs.at[src_v.at[j]], rows_v, sem).wait()

            def scale_body(g, _):
                wv = ew_v[j, pl.ds(g * 16, 16)]
                for r in range(16):
                    k = g * 16 + r
                    w = wv[r]
                    for t in range(nvec):
                        sl = pl.ds(t * 16, 16)
                        rows_v[k, sl] = rows_v[k, sl] * w
                return _

            lax.fori_loop(0, CHUNK // 16, scale_body, None)
            pltpu.sync_copy(rows_v, acc.at[dst_v.at[j]], add=True)
            return _

        lax.fori_loop(0, NCHUNK, chunk_body, None)
        plsc.subcore_barrier()
        pltpu.sync_copy(acc.at[stripe], out.at[cid, stripe])

    return agg


_sc_agg64 = _make_sc_aggregate(64)
_sc_agg16 = _make_sc_aggregate(16)


_BN = 1000


def _tc_prep_body(degp_ref, x_ref, w1_ref, dinv_ref, hs1_ref):
    deg = degp_ref[:, 0] + degp_ref[:, 1] + 1.0
    dv = lax.rsqrt(deg)
    h = jnp.dot(x_ref[...], w1_ref[...], preferred_element_type=jnp.float32)
    dinv_ref[...] = dv[:, None]
    hs1_ref[...] = h * dv[:, None]


def _tc_prep(degp, x, w1):
    return pl.pallas_call(
        _tc_prep_body,
        grid=(N // _BN,),
        in_specs=[
            pl.BlockSpec((_BN, 2), lambda i: (i, 0)),
            pl.BlockSpec((_BN, 128), lambda i: (i, 0)),
            pl.BlockSpec((128, 64), lambda i: (0, 0)),
        ],
        out_specs=[
            pl.BlockSpec((_BN, 1), lambda i: (i, 0)),
            pl.BlockSpec((_BN, 64), lambda i: (i, 0)),
        ],
        out_shape=[
            jax.ShapeDtypeStruct((N, 1), jnp.float32),
            jax.ShapeDtypeStruct((N, 64), jnp.float32),
        ],
    )(degp, x, w1)


def _tc_mid_body(s1_ref, hs1_ref, dinv_ref, b1_ref, w2_ref, hs2_ref):
    s = s1_ref[0] + s1_ref[1] + hs1_ref[...]
    dv = dinv_ref[...]
    h1a = jnp.maximum(s * dv + b1_ref[...], 0.0)
    h2 = jnp.dot(h1a, w2_ref[...], preferred_element_type=jnp.float32)
    hs2_ref[...] = h2 * dv


def _tc_mid(s1p, hs1, dinv, b1, w2p):
    return pl.pallas_call(
        _tc_mid_body,
        grid=(N // _BN,),
        in_specs=[
            pl.BlockSpec((2, _BN, 64), lambda i: (0, i, 0)),
            pl.BlockSpec((_BN, 64), lambda i: (i, 0)),
            pl.BlockSpec((_BN, 1), lambda i: (i, 0)),
            pl.BlockSpec((1, 64), lambda i: (0, 0)),
            pl.BlockSpec((64, 16), lambda i: (0, 0)),
        ],
        out_specs=pl.BlockSpec((_BN, 16), lambda i: (i, 0)),
        out_shape=jax.ShapeDtypeStruct((N, 16), jnp.float32),
    )(s1p, hs1, dinv, b1, w2p)


def _tc_final_body(s2_ref, hs2_ref, dinv_ref, b2_ref, out_ref):
    s = s2_ref[0] + s2_ref[1] + hs2_ref[...]
    out_ref[...] = s * dinv_ref[...] + b2_ref[...]


def _tc_final(s2p, hs2, dinv, b2p):
    return pl.pallas_call(
        _tc_final_body,
        grid=(N // _BN,),
        in_specs=[
            pl.BlockSpec((2, _BN, 16), lambda i: (0, i, 0)),
            pl.BlockSpec((_BN, 16), lambda i: (i, 0)),
            pl.BlockSpec((_BN, 1), lambda i: (i, 0)),
            pl.BlockSpec((1, 16), lambda i: (0, 0)),
        ],
        out_specs=pl.BlockSpec((_BN, 16), lambda i: (i, 0)),
        out_shape=jax.ShapeDtypeStruct((N, 16), jnp.float32),
    )(s2p, hs2, dinv, b2p)


def kernel(x, edge_index, edge_weight, W1, b1, W2, b2):
    ep = NTILES * EPT
    src = jnp.zeros((ep,), jnp.int32).at[:E].set(edge_index[0])
    dst = jnp.zeros((ep,), jnp.int32).at[:E].set(edge_index[1])
    ew = jnp.zeros((ep,), jnp.float32).at[:E].set(edge_weight)
    srcp = src.reshape(NTILES, NCHUNK, CHUNK)
    dstp = dst.reshape(NTILES, NCHUNK, CHUNK)
    ewp = ew.reshape(NTILES, NCHUNK, CHUNK)

    zeros1d = jnp.zeros((N_PAD,), jnp.float32)
    zeros64 = jnp.zeros((N_PAD, 64), jnp.float32)
    zeros16 = jnp.zeros((N_PAD, 16), jnp.float32)

    degp = _sc_degree(dstp, ewp, zeros1d)[:, :N].T
    dinv, hs1 = _tc_prep(degp, x, W1)

    s1p = _sc_agg64(hs1, srcp, dstp, ewp, zeros64)[:, :N]

    w2p = jnp.zeros((64, 16), jnp.float32).at[:, :3].set(W2)
    b1r = b1.reshape(1, 64)
    hs2 = _tc_mid(s1p, hs1, dinv, b1r, w2p)

    s2p = _sc_agg16(hs2, srcp, dstp, ewp, zeros16)[:, :N]

    b2p = jnp.zeros((1, 16), jnp.float32).at[0, :3].set(b2)
    beta = _tc_final(s2p, hs2, dinv, b2p)
    return beta[:, :3]


# speedup vs baseline: 23.0891x; 1.5228x over previous
import functools

import jax
import jax.numpy as jnp
from jax import lax
from jax.experimental import pallas as pl
from jax.experimental.pallas import tpu as pltpu
from jax.experimental.pallas import tpu_sc as plsc

N = 10000
E = 320000
NTILES = 32
CHUNK = 128
NCHUNK = 80
EPT = CHUNK * NCHUNK
N_PAD = 10240

_mesh = plsc.VectorSubcoreMesh(core_axis_name="c", subcore_axis_name="s")
_sc_params = pltpu.CompilerParams(use_tc_tiling_on_sc=False)


@functools.partial(
    pl.kernel,
    out_type=jax.ShapeDtypeStruct((2, N_PAD), jnp.float32),
    mesh=_mesh,
    compiler_params=_sc_params,
    scratch_types=[
        pltpu.VMEM((NCHUNK, CHUNK), jnp.int32),
        pltpu.VMEM((NCHUNK, CHUNK), jnp.float32),
        pltpu.VMEM_SHARED((N_PAD,), jnp.float32),
    ],
)
def _sc_degree(dstp, ewp, zeros1d, out, dst_v, ew_v, acc):
    cid = lax.axis_index("c")
    sid = lax.axis_index("s")
    wid = cid * 16 + sid
    stripe = pl.ds(sid * 640, 640)
    pltpu.sync_copy(zeros1d.at[stripe], acc.at[stripe])
    pltpu.sync_copy(dstp.at[wid], dst_v)
    pltpu.sync_copy(ewp.at[wid], ew_v)
    plsc.subcore_barrier()

    def body(j, _):
        pltpu.sync_copy(ew_v.at[j], acc.at[dst_v.at[j]], add=True)
        return _

    lax.fori_loop(0, NCHUNK, body, None)
    plsc.subcore_barrier()
    pltpu.sync_copy(acc.at[stripe], out.at[cid, stripe])


def _make_sc_aggregate(d):
    nvec = d // 16

    nbuf = 4

    @functools.partial(
        pl.kernel,
        out_type=jax.ShapeDtypeStruct((2, N_PAD, d), jnp.float32),
        mesh=_mesh,
        compiler_params=_sc_params,
        scratch_types=[
            pltpu.VMEM((NCHUNK, CHUNK), jnp.int32),
            pltpu.VMEM((NCHUNK, CHUNK), jnp.int32),
            pltpu.VMEM((NCHUNK, CHUNK), jnp.float32),
            [pltpu.VMEM((CHUNK, d), jnp.float32)] * nbuf,
            pltpu.VMEM_SHARED((N_PAD, d), jnp.float32),
            [pltpu.SemaphoreType.DMA] * nbuf,
            [pltpu.SemaphoreType.DMA] * nbuf,
        ],
    )
    def agg(hs, srcp, dstp, ewp, zeros2d, out, src_v, dst_v, ew_v, rows,
            acc, gsem, ssem):
        cid = lax.axis_index("c")
        sid = lax.axis_index("s")
        wid = cid * 16 + sid
        stripe = pl.ds(sid * 640, 640)
        pltpu.sync_copy(zeros2d.at[stripe], acc.at[stripe])
        pltpu.sync_copy(srcp.at[wid], src_v)
        pltpu.sync_copy(dstp.at[wid], dst_v)
        pltpu.sync_copy(ewp.at[wid], ew_v)
        plsc.subcore_barrier()

        def scale(buf, c):
            def scale_body(g, _):
                wv = ew_v[c, pl.ds(g * 16, 16)]
                for r in range(16):
                    k = g * 16 + r
                    w = wv[r]
                    for t in range(nvec):
                        sl = pl.ds(t * 16, 16)
                        buf[k, sl] = buf[k, sl] * w
                return _

            lax.fori_loop(0, CHUNK // 16, scale_body, None)

        for b in range(2):
            pltpu.async_copy(hs.at[src_v.at[b]], rows[b], gsem[b])

        def body(i, _):
            for b in range(nbuf):
                c = i * nbuf + b
                pltpu.make_async_copy(hs.at[src_v.at[c]], rows[b],
                                      gsem[b]).wait()
                scale(rows[b], c)
                pltpu.async_copy(rows[b], acc.at[dst_v.at[c]], ssem[b],
                                 add=True)
                b2 = (b + 2) % nbuf
                c2 = c + 2
                if b >= 2:
                    @pl.when(i < NCHUNK // nbuf - 1)
                    def _():
                        pltpu.make_async_copy(
                            rows[b2], acc.at[dst_v.at[c2 - nbuf]],
                            ssem[b2]).wait()
                        pltpu.async_copy(hs.at[src_v.at[c2]], rows[b2],
                                         gsem[b2])
                else:
                    @pl.when(i >= 1)
                    def _():
                        pltpu.make_async_copy(
                            rows[b2], acc.at[dst_v.at[c2 - nbuf]],
                            ssem[b2]).wait()

                    pltpu.async_copy(hs.at[src_v.at[c2]], rows[b2],
                                     gsem[b2])
            return _

        lax.fori_loop(0, NCHUNK // nbuf, body, None)
        for b in range(nbuf):
            c = NCHUNK - nbuf + b
            pltpu.make_async_copy(rows[b], acc.at[dst_v.at[c]],
                                  ssem[b]).wait()
        plsc.subcore_barrier()
        pltpu.sync_copy(acc.at[stripe], out.at[cid, stripe])

    return agg


_sc_agg64 = _make_sc_aggregate(64)
_sc_agg16 = _make_sc_aggregate(16)


_BN = 1000


def _tc_prep_body(degp_ref, x_ref, w1_ref, dinv_ref, hs1_ref):
    deg = degp_ref[:, 0] + degp_ref[:, 1] + 1.0
    dv = lax.rsqrt(deg)
    h = jnp.dot(x_ref[...], w1_ref[...], preferred_element_type=jnp.float32)
    dinv_ref[...] = dv[:, None]
    hs1_ref[...] = h * dv[:, None]


def _tc_prep(degp, x, w1):
    return pl.pallas_call(
        _tc_prep_body,
        grid=(N // _BN,),
        in_specs=[
            pl.BlockSpec((_BN, 2), lambda i: (i, 0)),
            pl.BlockSpec((_BN, 128), lambda i: (i, 0)),
            pl.BlockSpec((128, 64), lambda i: (0, 0)),
        ],
        out_specs=[
            pl.BlockSpec((_BN, 1), lambda i: (i, 0)),
            pl.BlockSpec((_BN, 64), lambda i: (i, 0)),
        ],
        out_shape=[
            jax.ShapeDtypeStruct((N, 1), jnp.float32),
            jax.ShapeDtypeStruct((N, 64), jnp.float32),
        ],
    )(degp, x, w1)


def _tc_mid_body(s1_ref, hs1_ref, dinv_ref, b1_ref, w2_ref, hs2_ref):
    s = s1_ref[0] + s1_ref[1] + hs1_ref[...]
    dv = dinv_ref[...]
    h1a = jnp.maximum(s * dv + b1_ref[...], 0.0)
    h2 = jnp.dot(h1a, w2_ref[...], preferred_element_type=jnp.float32)
    hs2_ref[...] = h2 * dv


def _tc_mid(s1p, hs1, dinv, b1, w2p):
    return pl.pallas_call(
        _tc_mid_body,
        grid=(N // _BN,),
        in_specs=[
            pl.BlockSpec((2, _BN, 64), lambda i: (0, i, 0)),
            pl.BlockSpec((_BN, 64), lambda i: (i, 0)),
            pl.BlockSpec((_BN, 1), lambda i: (i, 0)),
            pl.BlockSpec((1, 64), lambda i: (0, 0)),
            pl.BlockSpec((64, 16), lambda i: (0, 0)),
        ],
        out_specs=pl.BlockSpec((_BN, 16), lambda i: (i, 0)),
        out_shape=jax.ShapeDtypeStruct((N, 16), jnp.float32),
    )(s1p, hs1, dinv, b1, w2p)


def _tc_final_body(s2_ref, hs2_ref, dinv_ref, b2_ref, out_ref):
    s = s2_ref[0] + s2_ref[1] + hs2_ref[...]
    out_ref[...] = s * dinv_ref[...] + b2_ref[...]


def _tc_final(s2p, hs2, dinv, b2p):
    return pl.pallas_call(
        _tc_final_body,
        grid=(N // _BN,),
        in_specs=[
            pl.BlockSpec((2, _BN, 16), lambda i: (0, i, 0)),
            pl.BlockSpec((_BN, 16), lambda i: (i, 0)),
            pl.BlockSpec((_BN, 1), lambda i: (i, 0)),
            pl.BlockSpec((1, 16), lambda i: (0, 0)),
        ],
        out_specs=pl.BlockSpec((_BN, 16), lambda i: (i, 0)),
        out_shape=jax.ShapeDtypeStruct((N, 16), jnp.float32),
    )(s2p, hs2, dinv, b2p)


def kernel(x, edge_index, edge_weight, W1, b1, W2, b2):
    ep = NTILES * EPT
    src = jnp.zeros((ep,), jnp.int32).at[:E].set(edge_index[0])
    dst = jnp.zeros((ep,), jnp.int32).at[:E].set(edge_index[1])
    ew = jnp.zeros((ep,), jnp.float32).at[:E].set(edge_weight)
    srcp = src.reshape(NTILES, NCHUNK, CHUNK)
    dstp = dst.reshape(NTILES, NCHUNK, CHUNK)
    ewp = ew.reshape(NTILES, NCHUNK, CHUNK)

    zeros1d = jnp.zeros((N_PAD,), jnp.float32)
    zeros64 = jnp.zeros((N_PAD, 64), jnp.float32)
    zeros16 = jnp.zeros((N_PAD, 16), jnp.float32)

    degp = _sc_degree(dstp, ewp, zeros1d)[:, :N].T
    dinv, hs1 = _tc_prep(degp, x, W1)

    s1p = _sc_agg64(hs1, srcp, dstp, ewp, zeros64)[:, :N]

    w2p = jnp.zeros((64, 16), jnp.float32).at[:, :3].set(W2)
    b1r = b1.reshape(1, 64)
    hs2 = _tc_mid(s1p, hs1, dinv, b1r, w2p)

    s2p = _sc_agg16(hs2, srcp, dstp, ewp, zeros16)[:, :N]

    b2p = jnp.zeros((1, 16), jnp.float32).at[0, :3].set(b2)
    beta = _tc_final(s2p, hs2, dinv, b2p)
    return beta[:, :3]


# speedup vs baseline: 23.2454x; 1.0068x over previous
import functools

import jax
import jax.numpy as jnp
from jax import lax
from jax.experimental import pallas as pl
from jax.experimental.pallas import tpu as pltpu
from jax.experimental.pallas import tpu_sc as plsc

N = 10000
E = 320000
NTILES = 32
CHUNK = 128
NCHUNK = 80
EPT = CHUNK * NCHUNK
N_PAD = 10240

_mesh = plsc.VectorSubcoreMesh(core_axis_name="c", subcore_axis_name="s")
_sc_params = pltpu.CompilerParams(use_tc_tiling_on_sc=False)


@functools.partial(
    pl.kernel,
    out_type=jax.ShapeDtypeStruct((2, N_PAD), jnp.float32),
    mesh=_mesh,
    compiler_params=_sc_params,
    scratch_types=[
        pltpu.VMEM((NCHUNK, CHUNK), jnp.int32),
        pltpu.VMEM((NCHUNK, CHUNK), jnp.float32),
        pltpu.VMEM_SHARED((N_PAD,), jnp.float32),
    ],
)
def _sc_degree(dstp, ewp, zeros1d, out, dst_v, ew_v, acc):
    cid = lax.axis_index("c")
    sid = lax.axis_index("s")
    wid = cid * 16 + sid
    stripe = pl.ds(sid * 640, 640)
    pltpu.sync_copy(zeros1d.at[stripe], acc.at[stripe])
    pltpu.sync_copy(dstp.at[wid], dst_v)
    pltpu.sync_copy(ewp.at[wid], ew_v)
    plsc.subcore_barrier()

    def body(j, _):
        pltpu.sync_copy(ew_v.at[j], acc.at[dst_v.at[j]], add=True)
        return _

    lax.fori_loop(0, NCHUNK, body, None)
    plsc.subcore_barrier()
    pltpu.sync_copy(acc.at[stripe], out.at[cid, stripe])


def _make_sc_aggregate(d):
    nvec = d // 16

    nbuf = 5

    @functools.partial(
        pl.kernel,
        out_type=jax.ShapeDtypeStruct((2, N_PAD, d), jnp.float32),
        mesh=_mesh,
        compiler_params=_sc_params,
        scratch_types=[
            pltpu.VMEM((NCHUNK, CHUNK), jnp.int32),
            pltpu.VMEM((NCHUNK, CHUNK), jnp.int32),
            pltpu.VMEM((NCHUNK, CHUNK), jnp.float32),
            [pltpu.VMEM((CHUNK, d), jnp.float32)] * nbuf,
            pltpu.VMEM_SHARED((N_PAD, d), jnp.float32),
            [pltpu.SemaphoreType.DMA] * nbuf,
            [pltpu.SemaphoreType.DMA] * nbuf,
        ],
    )
    def agg(hs, srcp, dstp, ewp, zeros2d, out, src_v, dst_v, ew_v, rows,
            acc, gsem, ssem):
        cid = lax.axis_index("c")
        sid = lax.axis_index("s")
        wid = cid * 16 + sid
        stripe = pl.ds(sid * 640, 640)
        pltpu.sync_copy(zeros2d.at[stripe], acc.at[stripe])
        pltpu.sync_copy(srcp.at[wid], src_v)
        pltpu.sync_copy(dstp.at[wid], dst_v)
        pltpu.sync_copy(ewp.at[wid], ew_v)
        plsc.subcore_barrier()

        def scale(buf, c):
            def scale_body(g, _):
                wv = ew_v[c, pl.ds(g * 16, 16)]
                for r in range(16):
                    k = g * 16 + r
                    w = wv[r]
                    for t in range(nvec):
                        sl = pl.ds(t * 16, 16)
                        buf[k, sl] = buf[k, sl] * w
                return _

            lax.fori_loop(0, CHUNK // 16, scale_body, None)

        depth = nbuf - 2
        for b in range(depth):
            pltpu.async_copy(hs.at[src_v.at[b]], rows[b], gsem[b])

        def body(i, _):
            for b in range(nbuf):
                c = i * nbuf + b
                pltpu.make_async_copy(hs.at[src_v.at[c]], rows[b],
                                      gsem[b]).wait()
                scale(rows[b], c)
                pltpu.async_copy(rows[b], acc.at[dst_v.at[c]], ssem[b],
                                 add=True)
                bD = (b + depth) % nbuf
                cD = c + depth
                if b < 2:
                    @pl.when(i >= 1)
                    def _():
                        pltpu.make_async_copy(
                            rows[bD], acc.at[dst_v.at[c - 2]],
                            ssem[bD]).wait()

                    pltpu.async_copy(hs.at[src_v.at[cD]], rows[bD],
                                     gsem[bD])
                else:
                    @pl.when(i < NCHUNK // nbuf - 1)
                    def _():
                        pltpu.make_async_copy(
                            rows[bD], acc.at[dst_v.at[c - 2]],
                            ssem[bD]).wait()
                        pltpu.async_copy(hs.at[src_v.at[cD]], rows[bD],
                                         gsem[bD])
            return _

        lax.fori_loop(0, NCHUNK // nbuf, body, None)
        for b in range(nbuf):
            c = NCHUNK - nbuf + b
            pltpu.make_async_copy(rows[b], acc.at[dst_v.at[c]],
                                  ssem[b]).wait()
        plsc.subcore_barrier()
        pltpu.sync_copy(acc.at[stripe], out.at[cid, stripe])

    return agg


_sc_agg64 = _make_sc_aggregate(64)
_sc_agg16 = _make_sc_aggregate(16)


_BN = 1000


def _tc_prep_body(degp_ref, x_ref, w1_ref, dinv_ref, hs1_ref):
    deg = degp_ref[:, 0] + degp_ref[:, 1] + 1.0
    dv = lax.rsqrt(deg)
    h = jnp.dot(x_ref[...], w1_ref[...], preferred_element_type=jnp.float32)
    dinv_ref[...] = dv[:, None]
    hs1_ref[...] = h * dv[:, None]


def _tc_prep(degp, x, w1):
    return pl.pallas_call(
        _tc_prep_body,
        grid=(N // _BN,),
        in_specs=[
            pl.BlockSpec((_BN, 2), lambda i: (i, 0)),
            pl.BlockSpec((_BN, 128), lambda i: (i, 0)),
            pl.BlockSpec((128, 64), lambda i: (0, 0)),
        ],
        out_specs=[
            pl.BlockSpec((_BN, 1), lambda i: (i, 0)),
            pl.BlockSpec((_BN, 64), lambda i: (i, 0)),
        ],
        out_shape=[
            jax.ShapeDtypeStruct((N, 1), jnp.float32),
            jax.ShapeDtypeStruct((N, 64), jnp.float32),
        ],
    )(degp, x, w1)


def _tc_mid_body(s1_ref, hs1_ref, dinv_ref, b1_ref, w2_ref, hs2_ref):
    s = s1_ref[0] + s1_ref[1] + hs1_ref[...]
    dv = dinv_ref[...]
    h1a = jnp.maximum(s * dv + b1_ref[...], 0.0)
    h2 = jnp.dot(h1a, w2_ref[...], preferred_element_type=jnp.float32)
    hs2_ref[...] = h2 * dv


def _tc_mid(s1p, hs1, dinv, b1, w2p):
    return pl.pallas_call(
        _tc_mid_body,
        grid=(N // _BN,),
        in_specs=[
            pl.BlockSpec((2, _BN, 64), lambda i: (0, i, 0)),
            pl.BlockSpec((_BN, 64), lambda i: (i, 0)),
            pl.BlockSpec((_BN, 1), lambda i: (i, 0)),
            pl.BlockSpec((1, 64), lambda i: (0, 0)),
            pl.BlockSpec((64, 16), lambda i: (0, 0)),
        ],
        out_specs=pl.BlockSpec((_BN, 16), lambda i: (i, 0)),
        out_shape=jax.ShapeDtypeStruct((N, 16), jnp.float32),
    )(s1p, hs1, dinv, b1, w2p)


def _tc_final_body(s2_ref, hs2_ref, dinv_ref, b2_ref, out_ref):
    s = s2_ref[0] + s2_ref[1] + hs2_ref[...]
    out_ref[...] = s * dinv_ref[...] + b2_ref[...]


def _tc_final(s2p, hs2, dinv, b2p):
    return pl.pallas_call(
        _tc_final_body,
        grid=(N // _BN,),
        in_specs=[
            pl.BlockSpec((2, _BN, 16), lambda i: (0, i, 0)),
            pl.BlockSpec((_BN, 16), lambda i: (i, 0)),
            pl.BlockSpec((_BN, 1), lambda i: (i, 0)),
            pl.BlockSpec((1, 16), lambda i: (0, 0)),
        ],
        out_specs=pl.BlockSpec((_BN, 16), lambda i: (i, 0)),
        out_shape=jax.ShapeDtypeStruct((N, 16), jnp.float32),
    )(s2p, hs2, dinv, b2p)


def kernel(x, edge_index, edge_weight, W1, b1, W2, b2):
    ep = NTILES * EPT
    src = jnp.zeros((ep,), jnp.int32).at[:E].set(edge_index[0])
    dst = jnp.zeros((ep,), jnp.int32).at[:E].set(edge_index[1])
    ew = jnp.zeros((ep,), jnp.float32).at[:E].set(edge_weight)
    srcp = src.reshape(NTILES, NCHUNK, CHUNK)
    dstp = dst.reshape(NTILES, NCHUNK, CHUNK)
    ewp = ew.reshape(NTILES, NCHUNK, CHUNK)

    zeros1d = jnp.zeros((N_PAD,), jnp.float32)
    zeros64 = jnp.zeros((N_PAD, 64), jnp.float32)
    zeros16 = jnp.zeros((N_PAD, 16), jnp.float32)

    degp = _sc_degree(dstp, ewp, zeros1d)[:, :N].T
    dinv, hs1 = _tc_prep(degp, x, W1)

    s1p = _sc_agg64(hs1, srcp, dstp, ewp, zeros64)[:, :N]

    w2p = jnp.zeros((64, 16), jnp.float32).at[:, :3].set(W2)
    b1r = b1.reshape(1, 64)
    hs2 = _tc_mid(s1p, hs1, dinv, b1r, w2p)

    s2p = _sc_agg16(hs2, srcp, dstp, ewp, zeros16)[:, :N]

    b2p = jnp.zeros((1, 16), jnp.float32).at[0, :3].set(b2)
    beta = _tc_final(s2p, hs2, dinv, b2p)
    return beta[:, :3]


# speedup vs baseline: 41.3507x; 1.7789x over previous
import functools

import jax
import jax.numpy as jnp
from jax import lax
from jax.experimental import pallas as pl
from jax.experimental.pallas import tpu as pltpu
from jax.experimental.pallas import tpu_sc as plsc

N = 10000
E = 320000
NTILES = 32
CHUNK = 128
NCHUNK = 80
EPT = CHUNK * NCHUNK
N_PAD = 10240

_mesh = plsc.VectorSubcoreMesh(core_axis_name="c", subcore_axis_name="s")
_sc_params = pltpu.CompilerParams(use_tc_tiling_on_sc=False, needs_layout_passes=False)


@functools.partial(
    pl.kernel,
    out_type=jax.ShapeDtypeStruct((2, N_PAD), jnp.float32),
    mesh=_mesh,
    compiler_params=_sc_params,
    scratch_types=[
        pltpu.VMEM((NCHUNK, CHUNK), jnp.int32),
        pltpu.VMEM((NCHUNK, CHUNK), jnp.float32),
        pltpu.VMEM_SHARED((N_PAD,), jnp.float32),
    ],
)
def _sc_degree(dstp, ewp, zeros1d, out, dst_v, ew_v, acc):
    cid = lax.axis_index("c")
    sid = lax.axis_index("s")
    wid = cid * 16 + sid
    stripe = pl.ds(sid * 640, 640)
    pltpu.sync_copy(zeros1d.at[stripe], acc.at[stripe])
    pltpu.sync_copy(dstp.at[wid], dst_v)
    pltpu.sync_copy(ewp.at[wid], ew_v)
    plsc.subcore_barrier()

    def body(j, _):
        pltpu.sync_copy(ew_v.at[j], acc.at[dst_v.at[j]], add=True)
        return _

    lax.fori_loop(0, NCHUNK, body, None)
    plsc.subcore_barrier()
    pltpu.sync_copy(acc.at[stripe], out.at[cid, stripe])


def _make_sc_aggregate(d):
    nvec = d // 16
    nbuf = 4
    nidx = 8

    @functools.partial(
        pl.kernel,
        out_type=jax.ShapeDtypeStruct((2, N_PAD, d), jnp.float32),
        mesh=_mesh,
        compiler_params=_sc_params,
        scratch_types=[
            [pltpu.VMEM((3, CHUNK), jnp.int32)] * nidx,
            [pltpu.VMEM((CHUNK, d), jnp.float32)] * nbuf,
            pltpu.VMEM_SHARED((N, d), jnp.float32),
            pltpu.VMEM_SHARED((N_PAD, d), jnp.float32),
            [pltpu.SemaphoreType.DMA] * nidx,
            [pltpu.SemaphoreType.DMA] * nbuf,
            [pltpu.SemaphoreType.DMA] * nbuf,
        ],
    )
    def agg(hs, epk, zeros2d, out, idxr, rows, table, acc, isem, gsem,
            ssem):
        cid = lax.axis_index("c")
        sid = lax.axis_index("s")
        wid = cid * 16 + sid
        stripe = pl.ds(sid * 640, 640)
        tstripe = pl.ds(sid * 625, 625)
        pltpu.sync_copy(zeros2d.at[stripe], acc.at[stripe])
        pltpu.sync_copy(hs.at[tstripe], table.at[tstripe])
        for q in range(nidx):
            pltpu.async_copy(epk.at[wid, q], idxr[q], isem[q])
        plsc.subcore_barrier()

        def scale(buf, q):
            def scale_body(g, _):
                wv = plsc.bitcast(idxr[q][2, pl.ds(g * 16, 16)],
                                  jnp.float32)
                for r in range(16):
                    k = g * 16 + r
                    w = wv[r]
                    for t in range(nvec):
                        sl = pl.ds(t * 16, 16)
                        buf[k, sl] = buf[k, sl] * w
                return _

            lax.fori_loop(0, CHUNK // 16, scale_body, None)

        for u in range(2):
            pltpu.make_async_copy(epk.at[wid, u], idxr[u], isem[u]).wait()
            pltpu.async_copy(table.at[idxr[u].at[0]], rows[u], gsem[u])

        niter = NCHUNK // nidx

        def body(i, _):
            for u in range(nidx):
                c = i * nidx + u
                b = u % nbuf
                pltpu.make_async_copy(table.at[idxr[u].at[0]], rows[b],
                                      gsem[b]).wait()
                scale(rows[b], u)
                pltpu.async_copy(rows[b], acc.at[idxr[u].at[1]], ssem[b],
                                 add=True)
                u2 = (u - 2) % nidx
                b2 = (u - 2) % nbuf

                def wait_prev_scatter():
                    pltpu.make_async_copy(rows[b2],
                                          acc.at[idxr[u2].at[1]],
                                          ssem[b2]).wait()

                def refill_idx():
                    pltpu.async_copy(epk.at[wid, c + 6], idxr[u2],
                                     isem[u2])

                def start_next_gather():
                    u1 = (u + 2) % nidx
                    b1 = (u + 2) % nbuf
                    pltpu.make_async_copy(epk.at[wid, c + 2], idxr[u1],
                                          isem[u1]).wait()
                    pltpu.async_copy(table.at[idxr[u1].at[0]], rows[b1],
                                     gsem[b1])

                if u < 2:
                    @pl.when(i >= 1)
                    def _():
                        wait_prev_scatter()
                        refill_idx()

                    start_next_gather()
                else:
                    wait_prev_scatter()
                    if u < 6:
                        @pl.when(i < niter - 1)
                        def _():
                            refill_idx()

                        start_next_gather()
                    else:
                        @pl.when(i < niter - 1)
                        def _():
                            refill_idx()
                            start_next_gather()
            return _

        lax.fori_loop(0, niter, body, None)
        for c in (NCHUNK - 2, NCHUNK - 1):
            pltpu.make_async_copy(rows[c % nbuf],
                                  acc.at[idxr[c % nidx].at[1]],
                                  ssem[c % nbuf]).wait()
        plsc.subcore_barrier()
        pltpu.sync_copy(acc.at[stripe], out.at[cid, stripe])

    return agg


_sc_agg64 = _make_sc_aggregate(64)
_sc_agg16 = _make_sc_aggregate(16)


_BN = 1000


def _tc_prep_body(degp_ref, x_ref, w1_ref, dinv_ref, hs1_ref):
    deg = degp_ref[:, 0] + degp_ref[:, 1] + 1.0
    dv = lax.rsqrt(deg)
    h = jnp.dot(x_ref[...], w1_ref[...], preferred_element_type=jnp.float32)
    dinv_ref[...] = dv[:, None]
    hs1_ref[...] = h * dv[:, None]


def _tc_prep(degp, x, w1):
    return pl.pallas_call(
        _tc_prep_body,
        grid=(N // _BN,),
        in_specs=[
            pl.BlockSpec((_BN, 2), lambda i: (i, 0)),
            pl.BlockSpec((_BN, 128), lambda i: (i, 0)),
            pl.BlockSpec((128, 64), lambda i: (0, 0)),
        ],
        out_specs=[
            pl.BlockSpec((_BN, 1), lambda i: (i, 0)),
            pl.BlockSpec((_BN, 64), lambda i: (i, 0)),
        ],
        out_shape=[
            jax.ShapeDtypeStruct((N, 1), jnp.float32),
            jax.ShapeDtypeStruct((N, 64), jnp.float32),
        ],
    )(degp, x, w1)


def _tc_mid_body(s1_ref, hs1_ref, dinv_ref, b1_ref, w2_ref, hs2_ref):
    s = s1_ref[0] + s1_ref[1] + hs1_ref[...]
    dv = dinv_ref[...]
    h1a = jnp.maximum(s * dv + b1_ref[...], 0.0)
    h2 = jnp.dot(h1a, w2_ref[...], preferred_element_type=jnp.float32)
    hs2_ref[...] = h2 * dv


def _tc_mid(s1p, hs1, dinv, b1, w2p):
    return pl.pallas_call(
        _tc_mid_body,
        grid=(N // _BN,),
        in_specs=[
            pl.BlockSpec((2, _BN, 64), lambda i: (0, i, 0)),
            pl.BlockSpec((_BN, 64), lambda i: (i, 0)),
            pl.BlockSpec((_BN, 1), lambda i: (i, 0)),
            pl.BlockSpec((1, 64), lambda i: (0, 0)),
            pl.BlockSpec((64, 16), lambda i: (0, 0)),
        ],
        out_specs=pl.BlockSpec((_BN, 16), lambda i: (i, 0)),
        out_shape=jax.ShapeDtypeStruct((N, 16), jnp.float32),
    )(s1p, hs1, dinv, b1, w2p)


def _tc_final_body(s2_ref, hs2_ref, dinv_ref, b2_ref, out_ref):
    s = s2_ref[0] + s2_ref[1] + hs2_ref[...]
    out_ref[...] = s * dinv_ref[...] + b2_ref[...]


def _tc_final(s2p, hs2, dinv, b2p):
    return pl.pallas_call(
        _tc_final_body,
        grid=(N // _BN,),
        in_specs=[
            pl.BlockSpec((2, _BN, 16), lambda i: (0, i, 0)),
            pl.BlockSpec((_BN, 16), lambda i: (i, 0)),
            pl.BlockSpec((_BN, 1), lambda i: (i, 0)),
            pl.BlockSpec((1, 16), lambda i: (0, 0)),
        ],
        out_specs=pl.BlockSpec((_BN, 16), lambda i: (i, 0)),
        out_shape=jax.ShapeDtypeStruct((N, 16), jnp.float32),
    )(s2p, hs2, dinv, b2p)


def kernel(x, edge_index, edge_weight, W1, b1, W2, b2):
    ep = NTILES * EPT
    src = jnp.zeros((ep,), jnp.int32).at[:E].set(edge_index[0])
    dst = jnp.zeros((ep,), jnp.int32).at[:E].set(edge_index[1])
    ew = jnp.zeros((ep,), jnp.float32).at[:E].set(edge_weight)
    dstp = dst.reshape(NTILES, NCHUNK, CHUNK)
    ewp = ew.reshape(NTILES, NCHUNK, CHUNK)
    epk = jnp.stack(
        [src.reshape(NTILES, NCHUNK, CHUNK),
         dstp,
         lax.bitcast_convert_type(ewp, jnp.int32)], axis=2)

    zeros1d = jnp.zeros((N_PAD,), jnp.float32)
    zeros64 = jnp.zeros((N_PAD, 64), jnp.float32)
    zeros16 = jnp.zeros((N_PAD, 16), jnp.float32)

    degp = _sc_degree(dstp, ewp, zeros1d)[:, :N].T
    dinv, hs1 = _tc_prep(degp, x, W1)

    s1p = _sc_agg64(hs1, epk, zeros64)[:, :N]

    w2p = jnp.zeros((64, 16), jnp.float32).at[:, :3].set(W2)
    b1r = b1.reshape(1, 64)
    hs2 = _tc_mid(s1p, hs1, dinv, b1r, w2p)

    s2p = _sc_agg16(hs2, epk, zeros16)[:, :N]

    b2p = jnp.zeros((1, 16), jnp.float32).at[0, :3].set(b2)
    beta = _tc_final(s2p, hs2, dinv, b2p)
    return beta[:, :3]


# speedup vs baseline: 43.2413x; 1.0457x over previous
import functools

import jax
import jax.numpy as jnp
from jax import lax
from jax.experimental import pallas as pl
from jax.experimental.pallas import tpu as pltpu
from jax.experimental.pallas import tpu_sc as plsc

N = 10000
E = 320000
NTILES = 32
CHUNK = 128
NCHUNK = 80
EPT = CHUNK * NCHUNK
N_PAD = 10240

_mesh = plsc.VectorSubcoreMesh(core_axis_name="c", subcore_axis_name="s")
_sc_params = pltpu.CompilerParams(use_tc_tiling_on_sc=False, needs_layout_passes=False, skip_device_barrier=True)


@functools.partial(
    pl.kernel,
    out_type=jax.ShapeDtypeStruct((2, N_PAD), jnp.float32),
    mesh=_mesh,
    compiler_params=_sc_params,
    scratch_types=[
        pltpu.VMEM((NCHUNK, CHUNK), jnp.int32),
        pltpu.VMEM((NCHUNK, CHUNK), jnp.float32),
        pltpu.VMEM_SHARED((N_PAD,), jnp.float32),
        [pltpu.SemaphoreType.DMA] * 8,
    ],
)
def _sc_degree(dstp, ewp, zeros1d, out, dst_v, ew_v, acc, ssem):
    cid = lax.axis_index("c")
    sid = lax.axis_index("s")
    wid = cid * 16 + sid
    stripe = pl.ds(sid * 640, 640)
    pltpu.sync_copy(zeros1d.at[stripe], acc.at[stripe])
    pltpu.sync_copy(dstp.at[wid], dst_v)
    pltpu.sync_copy(ewp.at[wid], ew_v)
    plsc.subcore_barrier()

    for j in range(NCHUNK):
        if j >= 8:
            pltpu.make_async_copy(ew_v.at[j - 8],
                                  acc.at[dst_v.at[j - 8]],
                                  ssem[j % 8]).wait()
        pltpu.async_copy(ew_v.at[j], acc.at[dst_v.at[j]], ssem[j % 8],
                         add=True)
    for j in range(NCHUNK - 8, NCHUNK):
        pltpu.make_async_copy(ew_v.at[j], acc.at[dst_v.at[j]],
                              ssem[j % 8]).wait()
    plsc.subcore_barrier()
    pltpu.sync_copy(acc.at[stripe], out.at[cid, stripe])


def _make_sc_aggregate(d):
    nvec = d // 16
    nbuf = 4
    nidx = 8

    @functools.partial(
        pl.kernel,
        out_type=jax.ShapeDtypeStruct((2, N_PAD, d), jnp.float32),
        mesh=_mesh,
        compiler_params=_sc_params,
        scratch_types=[
            [pltpu.VMEM((3, CHUNK), jnp.int32)] * nidx,
            [pltpu.VMEM((CHUNK, d), jnp.float32)] * nbuf,
            pltpu.VMEM_SHARED((N, d), jnp.float32),
            pltpu.VMEM_SHARED((N_PAD, d), jnp.float32),
            [pltpu.SemaphoreType.DMA] * nidx,
            [pltpu.SemaphoreType.DMA] * nbuf,
            [pltpu.SemaphoreType.DMA] * nbuf,
        ],
    )
    def agg(hs, epk, zeros2d, out, idxr, rows, table, acc, isem, gsem,
            ssem):
        cid = lax.axis_index("c")
        sid = lax.axis_index("s")
        wid = cid * 16 + sid
        stripe = pl.ds(sid * 640, 640)
        tstripe = pl.ds(sid * 625, 625)
        pltpu.sync_copy(zeros2d.at[stripe], acc.at[stripe])
        pltpu.sync_copy(hs.at[tstripe], table.at[tstripe])
        for q in range(nidx):
            pltpu.async_copy(epk.at[wid, q], idxr[q], isem[q])
        plsc.subcore_barrier()

        def scale(buf, q):
            def scale_body(g2, _):
                for h in range(2):
                    g = g2 * 2 + h
                    wv = plsc.bitcast(idxr[q][2, pl.ds(g * 16, 16)],
                                      jnp.float32)
                    for r in range(16):
                        k = g * 16 + r
                        w = wv[r]
                        for t in range(nvec):
                            sl = pl.ds(t * 16, 16)
                            buf[k, sl] = buf[k, sl] * w
                return _

            lax.fori_loop(0, CHUNK // 32, scale_body, None)

        for u in range(2):
            pltpu.make_async_copy(epk.at[wid, u], idxr[u], isem[u]).wait()
            pltpu.async_copy(table.at[idxr[u].at[0]], rows[u], gsem[u])

        niter = NCHUNK // nidx

        def body(i, _):
            for u in range(nidx):
                c = i * nidx + u
                b = u % nbuf
                pltpu.make_async_copy(table.at[idxr[u].at[0]], rows[b],
                                      gsem[b]).wait()
                scale(rows[b], u)
                pltpu.async_copy(rows[b], acc.at[idxr[u].at[1]], ssem[b],
                                 add=True)
                u2 = (u - 2) % nidx
                b2 = (u - 2) % nbuf

                def wait_prev_scatter():
                    pltpu.make_async_copy(rows[b2],
                                          acc.at[idxr[u2].at[1]],
                                          ssem[b2]).wait()

                def refill_idx():
                    pltpu.async_copy(epk.at[wid, c + 6], idxr[u2],
                                     isem[u2])

                def start_next_gather():
                    u1 = (u + 2) % nidx
                    b1 = (u + 2) % nbuf
                    pltpu.make_async_copy(epk.at[wid, c + 2], idxr[u1],
                                          isem[u1]).wait()
                    pltpu.async_copy(table.at[idxr[u1].at[0]], rows[b1],
                                     gsem[b1])

                if u < 2:
                    @pl.when(i >= 1)
                    def _():
                        wait_prev_scatter()
                        refill_idx()

                    start_next_gather()
                else:
                    wait_prev_scatter()
                    if u < 6:
                        @pl.when(i < niter - 1)
                        def _():
                            refill_idx()

                        start_next_gather()
                    else:
                        @pl.when(i < niter - 1)
                        def _():
                            refill_idx()
                            start_next_gather()
            return _

        lax.fori_loop(0, niter, body, None)
        for c in (NCHUNK - 2, NCHUNK - 1):
            pltpu.make_async_copy(rows[c % nbuf],
                                  acc.at[idxr[c % nidx].at[1]],
                                  ssem[c % nbuf]).wait()
        plsc.subcore_barrier()
        pltpu.sync_copy(acc.at[stripe], out.at[cid, stripe])

    return agg


_sc_agg64 = _make_sc_aggregate(64)
_sc_agg16 = _make_sc_aggregate(16)


_BN = 1000


def _tc_prep_body(degp_ref, x_ref, w1_ref, dinv_ref, hs1_ref):
    deg = degp_ref[:, 0] + degp_ref[:, 1] + 1.0
    dv = lax.rsqrt(deg)
    h = jnp.dot(x_ref[...], w1_ref[...], preferred_element_type=jnp.float32)
    dinv_ref[...] = dv[:, None]
    hs1_ref[...] = h * dv[:, None]


def _tc_prep(degp, x, w1):
    return pl.pallas_call(
        _tc_prep_body,
        grid=(N // _BN,),
        in_specs=[
            pl.BlockSpec((_BN, 2), lambda i: (i, 0)),
            pl.BlockSpec((_BN, 128), lambda i: (i, 0)),
            pl.BlockSpec((128, 64), lambda i: (0, 0)),
        ],
        out_specs=[
            pl.BlockSpec((_BN, 1), lambda i: (i, 0)),
            pl.BlockSpec((_BN, 64), lambda i: (i, 0)),
        ],
        out_shape=[
            jax.ShapeDtypeStruct((N, 1), jnp.float32),
            jax.ShapeDtypeStruct((N, 64), jnp.float32),
        ],
    )(degp, x, w1)


def _tc_mid_body(s1_ref, hs1_ref, dinv_ref, b1_ref, w2_ref, hs2_ref):
    s = s1_ref[0] + s1_ref[1] + hs1_ref[...]
    dv = dinv_ref[...]
    h1a = jnp.maximum(s * dv + b1_ref[...], 0.0)
    h2 = jnp.dot(h1a, w2_ref[...], preferred_element_type=jnp.float32)
    hs2_ref[...] = h2 * dv


def _tc_mid(s1p, hs1, dinv, b1, w2p):
    return pl.pallas_call(
        _tc_mid_body,
        grid=(N // _BN,),
        in_specs=[
            pl.BlockSpec((2, _BN, 64), lambda i: (0, i, 0)),
            pl.BlockSpec((_BN, 64), lambda i: (i, 0)),
            pl.BlockSpec((_BN, 1), lambda i: (i, 0)),
            pl.BlockSpec((1, 64), lambda i: (0, 0)),
            pl.BlockSpec((64, 16), lambda i: (0, 0)),
        ],
        out_specs=pl.BlockSpec((_BN, 16), lambda i: (i, 0)),
        out_shape=jax.ShapeDtypeStruct((N, 16), jnp.float32),
    )(s1p, hs1, dinv, b1, w2p)


def _tc_final_body(s2_ref, hs2_ref, dinv_ref, b2_ref, out_ref):
    s = s2_ref[0] + s2_ref[1] + hs2_ref[...]
    out_ref[...] = s * dinv_ref[...] + b2_ref[...]


def _tc_final(s2p, hs2, dinv, b2p):
    return pl.pallas_call(
        _tc_final_body,
        grid=(N // _BN,),
        in_specs=[
            pl.BlockSpec((2, _BN, 16), lambda i: (0, i, 0)),
            pl.BlockSpec((_BN, 16), lambda i: (i, 0)),
            pl.BlockSpec((_BN, 1), lambda i: (i, 0)),
            pl.BlockSpec((1, 16), lambda i: (0, 0)),
        ],
        out_specs=pl.BlockSpec((_BN, 16), lambda i: (i, 0)),
        out_shape=jax.ShapeDtypeStruct((N, 16), jnp.float32),
    )(s2p, hs2, dinv, b2p)


def kernel(x, edge_index, edge_weight, W1, b1, W2, b2):
    ep = NTILES * EPT
    src = jnp.zeros((ep,), jnp.int32).at[:E].set(edge_index[0])
    dst = jnp.zeros((ep,), jnp.int32).at[:E].set(edge_index[1])
    ew = jnp.zeros((ep,), jnp.float32).at[:E].set(edge_weight)
    dstp = dst.reshape(NTILES, NCHUNK, CHUNK)
    ewp = ew.reshape(NTILES, NCHUNK, CHUNK)
    epk = jnp.stack(
        [src.reshape(NTILES, NCHUNK, CHUNK),
         dstp,
         lax.bitcast_convert_type(ewp, jnp.int32)], axis=2)

    zeros1d = jnp.zeros((N_PAD,), jnp.float32)
    zeros64 = jnp.zeros((N_PAD, 64), jnp.float32)
    zeros16 = jnp.zeros((N_PAD, 16), jnp.float32)

    degp = _sc_degree(dstp, ewp, zeros1d)[:, :N].T
    dinv, hs1 = _tc_prep(degp, x, W1)

    s1p = _sc_agg64(hs1, epk, zeros64)

    w2p = jnp.zeros((64, 16), jnp.float32).at[:, :3].set(W2)
    b1r = b1.reshape(1, 64)
    hs2 = _tc_mid(s1p, hs1, dinv, b1r, w2p)

    s2p = _sc_agg16(hs2, epk, zeros16)

    b2p = jnp.zeros((1, 16), jnp.float32).at[0, :3].set(b2)
    beta = _tc_final(s2p, hs2, dinv, b2p)
    return beta[:, :3]


# speedup vs baseline: 43.2800x; 1.0009x over previous
import functools

import jax
import jax.numpy as jnp
from jax import lax
from jax.experimental import pallas as pl
from jax.experimental.pallas import tpu as pltpu
from jax.experimental.pallas import tpu_sc as plsc

N = 10000
E = 320000
NTILES = 32
CHUNK = 128
NCHUNK = 80
EPT = CHUNK * NCHUNK
N_PAD = 10240

_mesh = plsc.VectorSubcoreMesh(core_axis_name="c", subcore_axis_name="s",
                               num_cores=2, num_subcores=16)
_sc_params = pltpu.CompilerParams(use_tc_tiling_on_sc=False, needs_layout_passes=False, skip_device_barrier=True)


@functools.partial(
    pl.kernel,
    out_type=jax.ShapeDtypeStruct((2, N_PAD), jnp.float32),
    mesh=_mesh,
    compiler_params=_sc_params,
    scratch_types=[
        pltpu.VMEM((NCHUNK, CHUNK), jnp.int32),
        pltpu.VMEM((NCHUNK, CHUNK), jnp.float32),
        pltpu.VMEM_SHARED((N_PAD,), jnp.float32),
        [pltpu.SemaphoreType.DMA] * 8,
    ],
)
def _sc_degree(dstp, ewp, zeros1d, out, dst_v, ew_v, acc, ssem):
    cid = lax.axis_index("c")
    sid = lax.axis_index("s")
    wid = cid * 16 + sid
    stripe = pl.ds(sid * 640, 640)
    pltpu.sync_copy(zeros1d.at[stripe], acc.at[stripe])
    pltpu.sync_copy(dstp.at[wid], dst_v)
    pltpu.sync_copy(ewp.at[wid], ew_v)
    plsc.subcore_barrier()

    for j in range(NCHUNK):
        if j >= 8:
            pltpu.make_async_copy(ew_v.at[j - 8],
                                  acc.at[dst_v.at[j - 8]],
                                  ssem[j % 8]).wait()
        pltpu.async_copy(ew_v.at[j], acc.at[dst_v.at[j]], ssem[j % 8],
                         add=True)
    for j in range(NCHUNK - 8, NCHUNK):
        pltpu.make_async_copy(ew_v.at[j], acc.at[dst_v.at[j]],
                              ssem[j % 8]).wait()
    plsc.subcore_barrier()
    pltpu.sync_copy(acc.at[stripe], out.at[cid, stripe])


def _make_sc_aggregate(d):
    nvec = d // 16
    nbuf = 4
    nidx = 8

    @functools.partial(
        pl.kernel,
        out_type=jax.ShapeDtypeStruct((2, N_PAD, d), jnp.float32),
        mesh=_mesh,
        compiler_params=_sc_params,
        scratch_types=[
            [pltpu.VMEM((3, CHUNK), jnp.int32)] * nidx,
            [pltpu.VMEM((CHUNK, d), jnp.float32)] * nbuf,
            pltpu.VMEM_SHARED((N, d), jnp.float32),
            pltpu.VMEM_SHARED((N_PAD, d), jnp.float32),
            [pltpu.SemaphoreType.DMA] * nidx,
            [pltpu.SemaphoreType.DMA] * nbuf,
            [pltpu.SemaphoreType.DMA] * nbuf,
        ],
    )
    def agg(hs, epk, zeros2d, out, idxr, rows, table, acc, isem, gsem,
            ssem):
        cid = lax.axis_index("c")
        sid = lax.axis_index("s")
        wid = cid * 16 + sid
        stripe = pl.ds(sid * 640, 640)
        tstripe = pl.ds(sid * 625, 625)
        pltpu.sync_copy(zeros2d.at[stripe], acc.at[stripe])
        pltpu.sync_copy(hs.at[tstripe], table.at[tstripe])
        for q in range(nidx):
            pltpu.async_copy(epk.at[wid, q], idxr[q], isem[q])
        plsc.subcore_barrier()

        def scale(buf, q):
            def scale_body(g2, _):
                for h in range(2):
                    g = g2 * 2 + h
                    wv = plsc.bitcast(idxr[q][2, pl.ds(g * 16, 16)],
                                      jnp.float32)
                    for r in range(16):
                        k = g * 16 + r
                        w = wv[r]
                        for t in range(nvec):
                            sl = pl.ds(t * 16, 16)
                            buf[k, sl] = buf[k, sl] * w
                return _

            lax.fori_loop(0, CHUNK // 32, scale_body, None)

        for u in range(2):
            pltpu.make_async_copy(epk.at[wid, u], idxr[u], isem[u]).wait()
            pltpu.async_copy(table.at[idxr[u].at[0]], rows[u], gsem[u])

        niter = NCHUNK // nidx

        def body(i, _):
            for u in range(nidx):
                c = i * nidx + u
                b = u % nbuf
                pltpu.make_async_copy(table.at[idxr[u].at[0]], rows[b],
                                      gsem[b]).wait()
                scale(rows[b], u)
                pltpu.async_copy(rows[b], acc.at[idxr[u].at[1]], ssem[b],
                                 add=True)
                u2 = (u - 2) % nidx
                b2 = (u - 2) % nbuf

                def wait_prev_scatter():
                    pltpu.make_async_copy(rows[b2],
                                          acc.at[idxr[u2].at[1]],
                                          ssem[b2]).wait()

                def refill_idx():
                    pltpu.async_copy(epk.at[wid, c + 6], idxr[u2],
                                     isem[u2])

                def start_next_gather():
                    u1 = (u + 2) % nidx
                    b1 = (u + 2) % nbuf
                    pltpu.make_async_copy(epk.at[wid, c + 2], idxr[u1],
                                          isem[u1]).wait()
                    pltpu.async_copy(table.at[idxr[u1].at[0]], rows[b1],
                                     gsem[b1])

                if u < 2:
                    @pl.when(i >= 1)
                    def _():
                        wait_prev_scatter()
                        refill_idx()

                    start_next_gather()
                else:
                    wait_prev_scatter()
                    if u < 6:
                        @pl.when(i < niter - 1)
                        def _():
                            refill_idx()

                        start_next_gather()
                    else:
                        @pl.when(i < niter - 1)
                        def _():
                            refill_idx()
                            start_next_gather()
            return _

        lax.fori_loop(0, niter, body, None)
        for c in (NCHUNK - 2, NCHUNK - 1):
            pltpu.make_async_copy(rows[c % nbuf],
                                  acc.at[idxr[c % nidx].at[1]],
                                  ssem[c % nbuf]).wait()
        plsc.subcore_barrier()
        pltpu.sync_copy(acc.at[stripe], out.at[cid, stripe])

    return agg


_sc_agg64 = _make_sc_aggregate(64)
_sc_agg16 = _make_sc_aggregate(16)


_BN = 1000


def _tc_prep_body(degp_ref, x_ref, w1_ref, dinv_ref, hs1_ref):
    deg = degp_ref[:, 0] + degp_ref[:, 1] + 1.0
    dv = lax.rsqrt(deg)
    h = jnp.dot(x_ref[...], w1_ref[...], preferred_element_type=jnp.float32)
    dinv_ref[...] = dv[:, None]
    hs1_ref[...] = h * dv[:, None]


def _tc_prep(degp, x, w1):
    return pl.pallas_call(
        _tc_prep_body,
        grid=(N // _BN,),
        in_specs=[
            pl.BlockSpec((_BN, 2), lambda i: (i, 0)),
            pl.BlockSpec((_BN, 128), lambda i: (i, 0)),
            pl.BlockSpec((128, 64), lambda i: (0, 0)),
        ],
        out_specs=[
            pl.BlockSpec((_BN, 1), lambda i: (i, 0)),
            pl.BlockSpec((_BN, 64), lambda i: (i, 0)),
        ],
        out_shape=[
            jax.ShapeDtypeStruct((N, 1), jnp.float32),
            jax.ShapeDtypeStruct((N, 64), jnp.float32),
        ],
    )(degp, x, w1)


def _tc_mid_body(s1_ref, hs1_ref, dinv_ref, b1_ref, w2_ref, hs2_ref):
    s = s1_ref[0] + s1_ref[1] + hs1_ref[...]
    dv = dinv_ref[...]
    h1a = jnp.maximum(s * dv + b1_ref[...], 0.0)
    h2 = jnp.dot(h1a, w2_ref[...], preferred_element_type=jnp.float32)
    hs2_ref[...] = h2 * dv


def _tc_mid(s1p, hs1, dinv, b1, w2p):
    return pl.pallas_call(
        _tc_mid_body,
        grid=(N // _BN,),
        in_specs=[
            pl.BlockSpec((2, _BN, 64), lambda i: (0, i, 0)),
            pl.BlockSpec((_BN, 64), lambda i: (i, 0)),
            pl.BlockSpec((_BN, 1), lambda i: (i, 0)),
            pl.BlockSpec((1, 64), lambda i: (0, 0)),
            pl.BlockSpec((64, 16), lambda i: (0, 0)),
        ],
        out_specs=pl.BlockSpec((_BN, 16), lambda i: (i, 0)),
        out_shape=jax.ShapeDtypeStruct((N, 16), jnp.float32),
    )(s1p, hs1, dinv, b1, w2p)


def _tc_final_body(s2_ref, hs2_ref, dinv_ref, b2_ref, out_ref):
    s = s2_ref[0] + s2_ref[1] + hs2_ref[...]
    out_ref[...] = s * dinv_ref[...] + b2_ref[...]


def _tc_final(s2p, hs2, dinv, b2p):
    return pl.pallas_call(
        _tc_final_body,
        grid=(N // _BN,),
        in_specs=[
            pl.BlockSpec((2, _BN, 16), lambda i: (0, i, 0)),
            pl.BlockSpec((_BN, 16), lambda i: (i, 0)),
            pl.BlockSpec((_BN, 1), lambda i: (i, 0)),
            pl.BlockSpec((1, 16), lambda i: (0, 0)),
        ],
        out_specs=pl.BlockSpec((_BN, 16), lambda i: (i, 0)),
        out_shape=jax.ShapeDtypeStruct((N, 16), jnp.float32),
    )(s2p, hs2, dinv, b2p)


def kernel(x, edge_index, edge_weight, W1, b1, W2, b2):
    ep = NTILES * EPT
    src = jnp.zeros((ep,), jnp.int32).at[:E].set(edge_index[0])
    dst = jnp.zeros((ep,), jnp.int32).at[:E].set(edge_index[1])
    ew = jnp.zeros((ep,), jnp.float32).at[:E].set(edge_weight)
    dstp = dst.reshape(NTILES, NCHUNK, CHUNK)
    ewp = ew.reshape(NTILES, NCHUNK, CHUNK)
    epk = jnp.stack(
        [src.reshape(NTILES, NCHUNK, CHUNK),
         dstp,
         lax.bitcast_convert_type(ewp, jnp.int32)], axis=2)

    zeros1d = jnp.zeros((N_PAD,), jnp.float32)
    zeros64 = jnp.zeros((N_PAD, 64), jnp.float32)
    zeros16 = jnp.zeros((N_PAD, 16), jnp.float32)

    degp = _sc_degree(dstp, ewp, zeros1d)[:, :N].T
    dinv, hs1 = _tc_prep(degp, x, W1)

    s1p = _sc_agg64(hs1, epk, zeros64)

    w2p = jnp.zeros((64, 16), jnp.float32).at[:, :3].set(W2)
    b1r = b1.reshape(1, 64)
    hs2 = _tc_mid(s1p, hs1, dinv, b1r, w2p)

    s2p = _sc_agg16(hs2, epk, zeros16)

    b2p = jnp.zeros((1, 16), jnp.float32).at[0, :3].set(b2)
    beta = _tc_final(s2p, hs2, dinv, b2p)
    return beta[:, :3]
